# baseline (speedup 1.0000x reference)
import jax
import jax.numpy as jnp
from jax import lax
from jax.experimental import pallas as pl
from jax.experimental.pallas import tpu as pltpu
from jax.experimental.pallas import tpu_sc as plsc

E = 320000
F = 4
H = 128
L = 16
NC = 2
NS = 16
NW = NC * NS
TB = 128
NT = E // TB
KC = 6
NJ = -(-NT // NW)
NITER = -(-NJ // KC)
WB_BYTES = TB * H * 4


def _body(edge_hbm, table_hbm, w_hbm, b_hbm, out_hbm,
          ec, idx0, idx1, s_v, r0, w_v, b_v, esem, gsem, wsem):
    wid = lax.axis_index("s") * NC + lax.axis_index("c")

    pltpu.sync_copy(w_hbm, w_v)
    pltpu.sync_copy(b_hbm, b_v)
    w_regs = [w_v[pl.ds(h * L, L)] for h in range(H // L)]
    bb_regs = [b_v[pl.ds(h * L, L)] * 2.0 for h in range(H // L)]

    def fire_edge(tile, b):
        pltpu.async_copy(edge_hbm.at[tile],
                         ec.at[pl.ds(b * F, F), :], esem.at[b])

    def drain(src, dst, sem):
        pltpu.make_async_copy(src, dst, sem).wait()

    for b in range(KC):
        fire_edge(wid + NW * b, b)

    def ring_body(k, carry):
        j0 = k * KC
        for b in range(KC):
            tile = wid + NW * (j0 + b)

            @pl.when(tile < NT)
            def _(tile=tile, b=b):
                drain(edge_hbm.at[0], ec.at[pl.ds(b * F, F), :], esem.at[b])
                for t in range(TB // L):
                    sl = pl.ds(t * L, L)
                    dl = pl.ds(b * TB + t * L, L)
                    idx0[dl] = ec[b * F + 0, sl]
                    idx1[dl] = ec[b * F + 1, sl]
                    s_v[dl] = (ec[b * F + 2, sl]
                               + ec[b * F + 3, sl]).astype(jnp.float32)
                @pl.when(k > 0)
                def _(b=b):
                    drain(out_hbm.at[pl.ds(0, TB), :],
                          r0.at[pl.ds(b * TB, TB), :], wsem.at[b])
                r0b = r0.at[pl.ds(b * TB, TB), :]
                svb = s_v.at[pl.ds(b * TB, TB)]

                @plsc.parallel_loop(0, TB, 1, unroll=2)
                def fill_body(e, r0b=r0b, svb=svb):
                    sv = plsc.load_gather(
                        svb, [jnp.full((L,), e, jnp.int32)])
                    for h in range(H // L):
                        sl = pl.ds(h * L, L)
                        r0b[e, sl] = sv * w_regs[h] + bb_regs[h]

                pltpu.async_copy(table_hbm.at[idx0.at[pl.ds(b * TB, TB)]],
                                 r0b, gsem.at[b], add=True)
                pltpu.async_copy(table_hbm.at[idx1.at[pl.ds(b * TB, TB)]],
                                 r0b, gsem.at[b], add=True)

        for b in range(KC):
            tile_n = wid + NW * (j0 + KC + b)

            @pl.when(tile_n < NT)
            def _(tile_n=tile_n, b=b):
                fire_edge(tile_n, b)

        for b in range(KC):
            tile = wid + NW * (j0 + b)

            @pl.when(tile < NT)
            def _(tile=tile, b=b):
                r0b = r0.at[pl.ds(b * TB, TB), :]
                drain(table_hbm.at[idx0.at[pl.ds(b * TB, TB)]], r0b,
                      gsem.at[b])
                drain(table_hbm.at[idx1.at[pl.ds(b * TB, TB)]], r0b,
                      gsem.at[b])
                pltpu.async_copy(r0b, out_hbm.at[pl.ds(tile * TB, TB), :],
                                 wsem.at[b])
        return carry

    lax.fori_loop(0, NITER, ring_body, 0)

    for b in range(KC):
        drain(out_hbm.at[pl.ds(0, TB), :],
              r0.at[pl.ds(b * TB, TB), :], wsem.at[b])


@jax.jit
def _run(edge_t, emb_table, w_flat, lin_b):
    mesh = plsc.VectorSubcoreMesh(core_axis_name="c", subcore_axis_name="s")
    k = pl.kernel(
        _body,
        out_type=jax.ShapeDtypeStruct((E, H), jnp.float32),
        mesh=mesh,
        compiler_params=pltpu.CompilerParams(needs_layout_passes=False),
        scratch_types=[
            pltpu.VMEM((KC * F, TB), jnp.int32),
            pltpu.VMEM((KC * TB,), jnp.int32),
            pltpu.VMEM((KC * TB,), jnp.int32),
            pltpu.VMEM((KC * TB,), jnp.float32),
            pltpu.VMEM((KC * TB, H), jnp.float32),
            pltpu.VMEM((H,), jnp.float32),
            pltpu.VMEM((H,), jnp.float32),
            pltpu.SemaphoreType.DMA((KC,)),
            pltpu.SemaphoreType.DMA((KC,)),
            pltpu.SemaphoreType.DMA((KC,)),
        ],
    )
    return k(edge_t, emb_table, w_flat, lin_b)


def kernel(edge_feat, emb_table, lin_W, lin_b):
    edge_t = (edge_feat.astype(jnp.int32)
              .reshape(NT, TB, F).transpose(0, 2, 1))
    w_flat = lin_W.reshape(H)
    out = _run(edge_t, emb_table, w_flat, lin_b)
    return out.reshape(1, E, H)


# speedup vs baseline: 1.1766x; 1.1766x over previous
import jax
import jax.numpy as jnp
from jax import lax
from jax.experimental import pallas as pl
from jax.experimental.pallas import tpu as pltpu
from jax.experimental.pallas import tpu_sc as plsc

E = 320000
F = 4
H = 128
VOCAB = 10000
L = 16
NC = 2
NS = 16
NW = NC * NS
TB = 128
NT = E // TB
KC = 2
NJ = -(-NT // NW)
NITER = -(-NJ // KC)


def _body(edge_hbm, table_hbm, w_hbm, b_hbm, out_hbm,
          ec, idx0, idx1, s_v, r0, w_v, b_v, shared, esem, gsem, wsem):
    wid = lax.axis_index("s") * NC + lax.axis_index("c")
    sid = lax.axis_index("s")

    @pl.when(sid == 0)
    def _():
        pltpu.sync_copy(table_hbm, shared)

    plsc.subcore_barrier()

    pltpu.sync_copy(w_hbm, w_v)
    pltpu.sync_copy(b_hbm, b_v)
    w_regs = [w_v[pl.ds(h * L, L)] for h in range(H // L)]
    bb_regs = [b_v[pl.ds(h * L, L)] * 2.0 for h in range(H // L)]

    def fire_edge(tile, b):
        pltpu.async_copy(edge_hbm.at[tile],
                         ec.at[pl.ds(b * F, F), :], esem.at[b])

    def drain(src, dst, sem):
        pltpu.make_async_copy(src, dst, sem).wait()

    for b in range(KC):
        fire_edge(wid + NW * b, b)

    def ring_body(k, carry):
        j0 = k * KC
        for b in range(KC):
            tile = wid + NW * (j0 + b)

            @pl.when(tile < NT)
            def _(tile=tile, b=b):
                drain(edge_hbm.at[0], ec.at[pl.ds(b * F, F), :], esem.at[b])
                for t in range(TB // L):
                    sl = pl.ds(t * L, L)
                    dl = pl.ds(b * TB + t * L, L)
                    idx0[dl] = ec[b * F + 0, sl]
                    idx1[dl] = ec[b * F + 1, sl]
                    s_v[dl] = (ec[b * F + 2, sl]
                               + ec[b * F + 3, sl]).astype(jnp.float32)
                @pl.when(k > 0)
                def _(b=b):
                    drain(out_hbm.at[pl.ds(0, TB), :],
                          r0.at[pl.ds(b * TB, TB), :], wsem.at[b])
                r0b = r0.at[pl.ds(b * TB, TB), :]
                svb = s_v.at[pl.ds(b * TB, TB)]

                @plsc.parallel_loop(0, TB, 1, unroll=2)
                def fill_body(e, r0b=r0b, svb=svb):
                    sv = plsc.load_gather(
                        svb, [jnp.full((L,), e, jnp.int32)])
                    for h in range(H // L):
                        sl = pl.ds(h * L, L)
                        r0b[e, sl] = sv * w_regs[h] + bb_regs[h]

                pltpu.async_copy(shared.at[idx0.at[pl.ds(b * TB, TB)]],
                                 r0b, gsem.at[b], add=True)
                pltpu.async_copy(shared.at[idx1.at[pl.ds(b * TB, TB)]],
                                 r0b, gsem.at[b], add=True)

        for b in range(KC):
            tile_n = wid + NW * (j0 + KC + b)

            @pl.when(tile_n < NT)
            def _(tile_n=tile_n, b=b):
                fire_edge(tile_n, b)

        for b in range(KC):
            tile = wid + NW * (j0 + b)

            @pl.when(tile < NT)
            def _(tile=tile, b=b):
                r0b = r0.at[pl.ds(b * TB, TB), :]
                drain(shared.at[idx0.at[pl.ds(b * TB, TB)]], r0b,
                      gsem.at[b])
                drain(shared.at[idx1.at[pl.ds(b * TB, TB)]], r0b,
                      gsem.at[b])
                pltpu.async_copy(r0b, out_hbm.at[pl.ds(tile * TB, TB), :],
                                 wsem.at[b])
        return carry

    lax.fori_loop(0, NITER, ring_body, 0)

    for b in range(KC):
        drain(out_hbm.at[pl.ds(0, TB), :],
              r0.at[pl.ds(b * TB, TB), :], wsem.at[b])


@jax.jit
def _run(edge_t, emb_table, w_flat, lin_b):
    mesh = plsc.VectorSubcoreMesh(core_axis_name="c", subcore_axis_name="s")
    k = pl.kernel(
        _body,
        out_type=jax.ShapeDtypeStruct((E, H), jnp.float32),
        mesh=mesh,
        compiler_params=pltpu.CompilerParams(needs_layout_passes=False),
        scratch_types=[
            pltpu.VMEM((KC * F, TB), jnp.int32),
            pltpu.VMEM((KC * TB,), jnp.int32),
            pltpu.VMEM((KC * TB,), jnp.int32),
            pltpu.VMEM((KC * TB,), jnp.float32),
            pltpu.VMEM((KC * TB, H), jnp.float32),
            pltpu.VMEM((H,), jnp.float32),
            pltpu.VMEM((H,), jnp.float32),
            pltpu.VMEM_SHARED((VOCAB, H), jnp.float32),
            pltpu.SemaphoreType.DMA((KC,)),
            pltpu.SemaphoreType.DMA((KC,)),
            pltpu.SemaphoreType.DMA((KC,)),
        ],
    )
    return k(edge_t, emb_table, w_flat, lin_b)


def kernel(edge_feat, emb_table, lin_W, lin_b):
    edge_t = (edge_feat.astype(jnp.int32)
              .reshape(NT, TB, F).transpose(0, 2, 1))
    w_flat = lin_W.reshape(H)
    out = _run(edge_t, emb_table, w_flat, lin_b)
    return out.reshape(1, E, H)
